# baseline (speedup 1.0000x reference)
import functools

import jax
import jax.numpy as jnp
from jax import lax
from jax.experimental import pallas as pl
from jax.experimental.pallas import tpu as pltpu
from jax.experimental.pallas import tpu_sc as plsc

N = 8192
K = 9
NBR_PAD = 16
TILE = 128
T2 = 512
LEAKY = 0.1
EPS = 1e-5


def _lrelu(x):
    return jnp.where(x >= 0, x, LEAKY * x)


def _knn_body(xr_ref, xc_ref, idx_ref, d_ref):
    a = xr_ref[...]
    b = xc_ref[...]
    a0, a1, a2 = a[:, 0:1], a[:, 1:2], a[:, 2:3]
    b0, b1, b2 = b[0:1, :], b[1:2, :], b[2:3, :]
    prod = jnp.dot(a.astype(jnp.bfloat16), b.astype(jnp.bfloat16),
                   preferred_element_type=jnp.float32)
    sqr = a0 * a0 + a1 * a1 + a2 * a2
    sqc = b0 * b0 + b1 * b1 + b2 * b2
    d_ref[...] = -2.0 * prod + sqr + sqc

    CH = 128
    NCH = N // CH
    lane = jax.lax.broadcasted_iota(jnp.int32, (TILE, CH), 1)
    cols = []
    prev = None
    for j in range(K):
        if prev is None:
            def scan0(t, carry):
                V, T = carry
                c = d_ref[:, pl.ds(t * CH, CH)]
                take = c < V
                return jnp.minimum(V, c), jnp.where(take, t, T)
            V = d_ref[:, 0:CH]
            T = jnp.zeros((TILE, CH), jnp.int32)
            V, T = jax.lax.fori_loop(1, NCH, scan0, (V, T))
        else:
            pv = prev

            def scanm(t, carry):
                V, T = carry
                c = d_ref[:, pl.ds(t * CH, CH)]
                c = jnp.where(lane + (t * CH) == pv, jnp.float32(jnp.inf), c)
                d_ref[:, pl.ds(t * CH, CH)] = c
                take = c < V
                return jnp.minimum(V, c), jnp.where(take, t, T)
            c0 = jnp.where(lane == pv, jnp.float32(jnp.inf), d_ref[:, 0:CH])
            d_ref[:, 0:CH] = c0
            V = c0
            T = jnp.zeros((TILE, CH), jnp.int32)
            V, T = jax.lax.fori_loop(1, NCH, scanm, (V, T))
        m = jnp.min(V, axis=1, keepdims=True)
        gidx = T * CH + lane
        idxv = jnp.min(jnp.where(V == m, gidx, jnp.int32(N)), axis=1,
                       keepdims=True)
        cols.append(idxv)
        prev = idxv
    cols.append(jnp.zeros((TILE, NBR_PAD - K), jnp.int32))
    idx_ref[...] = jnp.concatenate(cols, axis=1)


def _knn(xr, xc):
    return pl.pallas_call(
        _knn_body,
        grid=(N // TILE,),
        in_specs=[
            pl.BlockSpec((TILE, 3), lambda i: (i, 0)),
            pl.BlockSpec((3, N), lambda i: (0, 0)),
        ],
        out_specs=pl.BlockSpec((TILE, NBR_PAD), lambda i: (i, 0)),
        out_shape=jax.ShapeDtypeStruct((N, NBR_PAD), jnp.int32),
        scratch_shapes=[pltpu.VMEM((TILE, N), jnp.float32)],
    )(xr, xc)


def _sc_gather(table, idx2d, nrows, dcols):
    nw, ch = 32, 128
    per_w = nrows // nw
    n_ch = per_w // ch
    mesh = plsc.VectorSubcoreMesh(core_axis_name="c", subcore_axis_name="s")

    @functools.partial(
        pl.kernel, mesh=mesh,
        out_type=jax.ShapeDtypeStruct((nrows, dcols), jnp.float32),
        scratch_types=[
            pltpu.VMEM((n_ch, ch), jnp.int32),
            pltpu.VMEM((ch, dcols), jnp.float32),
            pltpu.SemaphoreType.DMA,
        ],
    )
    def k(table_hbm, idx_hbm, out_hbm, idx_v, rows_v, sem):
        wid = lax.axis_index("s") * 2 + lax.axis_index("c")
        pltpu.sync_copy(idx_hbm.at[pl.ds(wid * n_ch, n_ch)], idx_v)

        def body(i, carry):
            pltpu.async_copy(table_hbm.at[idx_v.at[i]], rows_v, sem).wait()
            pltpu.sync_copy(rows_v,
                            out_hbm.at[pl.ds(wid * per_w + i * ch, ch)])
            return carry
        jax.lax.fori_loop(0, n_ch, body, 0)

    return k(table, idx2d)


def _pconv_body(idx_ref, xyzpad_ref, table_ref, w0_ref, b0_ref, w1_ref,
                b1_ref, w2_ref, b2_ref, lin3_ref, linb_ref,
                pre_ref, stats_ref, g_ref, *, dpad, cout):
    def loadrow(n, carry):
        for k in range(K):
            iv = idx_ref[n, k]
            g_ref[pl.ds(k * TILE + n, 1), :] = table_ref[pl.ds(iv, 1), :]
        return carry
    jax.lax.fori_loop(0, TILE, loadrow, 0)

    xyz9 = jnp.concatenate([xyzpad_ref[...]] * K, axis=0)
    g_ref[...] = g_ref[...] - xyz9

    mask8 = (jax.lax.broadcasted_iota(jnp.int32, (1, 8), 1) < 3)
    g8 = jnp.where(mask8, g_ref[:, 0:8], 0.0)
    h = jnp.maximum(
        jnp.dot(g8, w0_ref[...], preferred_element_type=jnp.float32)
        + b0_ref[...], 0.0)
    h = jnp.maximum(
        jnp.dot(h, w1_ref[...], preferred_element_type=jnp.float32)
        + b1_ref[...], 0.0)
    w = jnp.maximum(
        jnp.dot(h, w2_ref[...], preferred_element_type=jnp.float32)
        + b2_ref[...], 0.0)

    pre = linb_ref[...]
    for wch in range(16):
        accw = jnp.zeros((TILE, dpad), jnp.float32)
        for k in range(K):
            accw = accw + (w[k * TILE:(k + 1) * TILE, wch:wch + 1]
                           * g_ref[k * TILE:(k + 1) * TILE, :])
        pre = pre + jnp.dot(accw, lin3_ref[wch],
                            preferred_element_type=jnp.float32)
    pre_ref[...] = pre

    @pl.when(pl.program_id(0) == 0)
    def _():
        stats_ref[...] = jnp.zeros_like(stats_ref)
    stats_ref[0:1, :] = stats_ref[0:1, :] + jnp.sum(pre, axis=0, keepdims=True)
    stats_ref[1:2, :] = stats_ref[1:2, :] + jnp.sum(pre * pre, axis=0,
                                                    keepdims=True)


def _pconv(idx, xyzpad, table, w0, b0, w1, b1, w2, b2, lin3, linb, dpad, cout):
    return pl.pallas_call(
        functools.partial(_pconv_body, dpad=dpad, cout=cout),
        grid=(N // TILE,),
        in_specs=[
            pl.BlockSpec((TILE, NBR_PAD), lambda i: (i, 0),
                         memory_space=pltpu.SMEM),
            pl.BlockSpec((TILE, dpad), lambda i: (i, 0)),
            pl.BlockSpec((N, dpad), lambda i: (0, 0)),
            pl.BlockSpec((8, 8), lambda i: (0, 0)),
            pl.BlockSpec((1, 8), lambda i: (0, 0)),
            pl.BlockSpec((8, 8), lambda i: (0, 0)),
            pl.BlockSpec((1, 8), lambda i: (0, 0)),
            pl.BlockSpec((8, 16), lambda i: (0, 0)),
            pl.BlockSpec((1, 16), lambda i: (0, 0)),
            pl.BlockSpec((16, dpad, cout), lambda i: (0, 0, 0)),
            pl.BlockSpec((1, cout), lambda i: (0, 0)),
        ],
        out_specs=[
            pl.BlockSpec((TILE, cout), lambda i: (i, 0)),
            pl.BlockSpec((8, cout), lambda i: (0, 0)),
        ],
        out_shape=[
            jax.ShapeDtypeStruct((N, cout), jnp.float32),
            jax.ShapeDtypeStruct((8, cout), jnp.float32),
        ],
        scratch_shapes=[pltpu.VMEM((TILE * K, dpad), jnp.float32)],
    )(idx, xyzpad, table, w0, b0, w1, b1, w2, b2, lin3, linb)


def _bn_body(pre_ref, stats_ref, g_ref, b_ref, out_ref):
    m = stats_ref[0:1, :] / N
    v = stats_ref[1:2, :] / N - m * m
    scale = g_ref[...] * jax.lax.rsqrt(v + EPS)
    out_ref[...] = _lrelu((pre_ref[...] - m) * scale + b_ref[...])


def _bn(pre, stats, gamma, beta, cout):
    return pl.pallas_call(
        _bn_body,
        grid=(N // T2,),
        in_specs=[
            pl.BlockSpec((T2, cout), lambda i: (i, 0)),
            pl.BlockSpec((8, cout), lambda i: (0, 0)),
            pl.BlockSpec((1, cout), lambda i: (0, 0)),
            pl.BlockSpec((1, cout), lambda i: (0, 0)),
        ],
        out_specs=pl.BlockSpec((T2, cout), lambda i: (i, 0)),
        out_shape=jax.ShapeDtypeStruct((N, cout), jnp.float32),
    )(pre, stats, gamma, beta)


def _tail_body(pre_ref, stats_ref, g_ref, b_ref, flow_ref, m0_ref, m0b_ref,
               m1_ref, m1b_ref, fc_ref, fcb_ref, np_ref, fl_ref):
    m = stats_ref[0:1, :] / N
    v = stats_ref[1:2, :] / N - m * m
    scale = g_ref[...] * jax.lax.rsqrt(v + EPS)
    x = _lrelu((pre_ref[...] - m) * scale + b_ref[...])
    h = _lrelu(jnp.dot(x, m0_ref[...], preferred_element_type=jnp.float32)
               + m0b_ref[...])
    h2 = _lrelu(jnp.dot(h, m1_ref[...], preferred_element_type=jnp.float32)
                + m1b_ref[...])
    fl = jnp.dot(h2, fc_ref[...], preferred_element_type=jnp.float32) \
        + fcb_ref[...]
    fl = jnp.clip(fl, -200.0, 200.0) + flow_ref[...]
    np_ref[...] = h2
    fl_ref[...] = fl


def _tail(pre, stats, gamma, beta, flowp, m0, m0b, m1, m1b, fc, fcb):
    return pl.pallas_call(
        _tail_body,
        grid=(N // T2,),
        in_specs=[
            pl.BlockSpec((T2, 128), lambda i: (i, 0)),
            pl.BlockSpec((8, 128), lambda i: (0, 0)),
            pl.BlockSpec((1, 128), lambda i: (0, 0)),
            pl.BlockSpec((1, 128), lambda i: (0, 0)),
            pl.BlockSpec((T2, 128), lambda i: (i, 0)),
            pl.BlockSpec((128, 128), lambda i: (0, 0)),
            pl.BlockSpec((1, 128), lambda i: (0, 0)),
            pl.BlockSpec((128, 64), lambda i: (0, 0)),
            pl.BlockSpec((1, 64), lambda i: (0, 0)),
            pl.BlockSpec((64, 128), lambda i: (0, 0)),
            pl.BlockSpec((1, 128), lambda i: (0, 0)),
        ],
        out_specs=[
            pl.BlockSpec((T2, 64), lambda i: (i, 0)),
            pl.BlockSpec((T2, 128), lambda i: (i, 0)),
        ],
        out_shape=[
            jax.ShapeDtypeStruct((N, 64), jnp.float32),
            jax.ShapeDtypeStruct((N, 128), jnp.float32),
        ],
    )(pre, stats, gamma, beta, flowp, m0, m0b, m1, m1b, fc, fcb)


def _prep_lin3(lin_w, d_in, dpad):
    c = lin_w.shape[0]
    l3 = lin_w.reshape(c, 3 + d_in, 16).transpose(2, 1, 0)
    return jnp.pad(l3, ((0, 0), (0, dpad - (3 + d_in)), (0, 0)))


def kernel(xyz, feats, cost_volume, flow, pc0_wn_w0, pc0_wn_b0, pc0_wn_w1,
           pc0_wn_b1, pc0_wn_w2, pc0_wn_b2, pc0_lin_w, pc0_lin_b, pc0_bn_g,
           pc0_bn_b, pc1_wn_w0, pc1_wn_b0, pc1_wn_w1, pc1_wn_b1, pc1_wn_w2,
           pc1_wn_b2, pc1_lin_w, pc1_lin_b, pc1_bn_g, pc1_bn_b, mlp0_w,
           mlp0_b, mlp1_w, mlp1_b, fc_w, fc_b):
    xc = xyz[0]
    xr = xc.T
    idx = _knn(xr, xc)

    pts0 = jnp.concatenate([feats, cost_volume], axis=1)[0].T
    d0, dpad0 = 192, 208
    table0 = jnp.pad(jnp.concatenate([xr, pts0], axis=1),
                     ((0, 0), (0, dpad0 - 3 - d0)))
    xyzpad0 = jnp.pad(xr, ((0, 0), (0, dpad0 - 3)))

    def wn_prep(w0, b0, w1, b1, w2, b2):
        w0p = jnp.zeros((8, 8), jnp.float32).at[0:3, :].set(w0.T)
        return (w0p, b0.reshape(1, 8), w1.T, b1.reshape(1, 8), w2.T,
                b2.reshape(1, 16))

    wn0 = wn_prep(pc0_wn_w0, pc0_wn_b0, pc0_wn_w1, pc0_wn_b1, pc0_wn_w2,
                  pc0_wn_b2)
    lin3_0 = _prep_lin3(pc0_lin_w, d0, dpad0)
    pre0, stats0 = _pconv(idx, xyzpad0, table0, *wn0, lin3_0,
                          pc0_lin_b.reshape(1, -1), dpad0, 128)
    pts1 = _bn(pre0, stats0, pc0_bn_g.reshape(1, -1),
               pc0_bn_b.reshape(1, -1), 128)

    d1, dpad1 = 128, 144
    table1 = jnp.pad(jnp.concatenate([xr, pts1], axis=1),
                     ((0, 0), (0, dpad1 - 3 - d1)))
    xyzpad1 = jnp.pad(xr, ((0, 0), (0, dpad1 - 3)))
    wn1 = wn_prep(pc1_wn_w0, pc1_wn_b0, pc1_wn_w1, pc1_wn_b1, pc1_wn_w2,
                  pc1_wn_b2)
    lin3_1 = _prep_lin3(pc1_lin_w, d1, dpad1)
    pre1, stats1 = _pconv(idx, xyzpad1, table1, *wn1, lin3_1,
                          pc1_lin_b.reshape(1, -1), dpad1, 128)

    flowp = jnp.pad(flow[0].T, ((0, 0), (0, 125)))
    fcp = jnp.pad(fc_w.T, ((0, 0), (0, 125)))
    fcbp = jnp.pad(fc_b, (0, 125)).reshape(1, 128)
    newp, flp = _tail(pre1, stats1, pc1_bn_g.reshape(1, -1),
                      pc1_bn_b.reshape(1, -1), flowp, mlp0_w.T,
                      mlp0_b.reshape(1, -1), mlp1_w.T, mlp1_b.reshape(1, -1),
                      fcp, fcbp)
    new_points = newp.T[None, :, :]
    flow_out = flp[:, 0:3].T[None, :, :]
    return new_points, flow_out


# speedup vs baseline: 1.2314x; 1.2314x over previous
import functools

import jax
import jax.numpy as jnp
from jax import lax
from jax.experimental import pallas as pl
from jax.experimental.pallas import tpu as pltpu
from jax.experimental.pallas import tpu_sc as plsc

N = 8192
K = 9
NBR_PAD = 16
TILE = 128
T2 = 512
LEAKY = 0.1
EPS = 1e-5


def _lrelu(x):
    return jnp.where(x >= 0, x, LEAKY * x)


def _knn_body(xr_ref, xc_ref, idx_ref, d_ref):
    a = xr_ref[...]
    b = xc_ref[...]
    a0, a1, a2 = a[:, 0:1], a[:, 1:2], a[:, 2:3]
    b0, b1, b2 = b[0:1, :], b[1:2, :], b[2:3, :]
    prod = jnp.dot(a.astype(jnp.bfloat16), b.astype(jnp.bfloat16),
                   preferred_element_type=jnp.float32)
    sqr = a0 * a0 + a1 * a1 + a2 * a2
    sqc = b0 * b0 + b1 * b1 + b2 * b2
    d_ref[...] = -2.0 * prod + sqr + sqc

    CH = 128
    NCH = N // CH
    lane = jax.lax.broadcasted_iota(jnp.int32, (TILE, CH), 1)
    cols = []
    prev = None
    for _ in range(K):
        V = None
        for t in range(NCH):
            c = d_ref[:, t * CH:(t + 1) * CH]
            if prev is not None:
                c = jnp.where(lane + (t * CH) == prev, jnp.float32(jnp.inf),
                              c)
                d_ref[:, t * CH:(t + 1) * CH] = c
            if V is None:
                V = c
                T = jnp.zeros((TILE, CH), jnp.int32)
            else:
                take = c < V
                V = jnp.minimum(V, c)
                T = jnp.where(take, jnp.int32(t), T)
        m = jnp.min(V, axis=1, keepdims=True)
        gidx = T * CH + lane
        idxv = jnp.min(jnp.where(V == m, gidx, jnp.int32(N)), axis=1,
                       keepdims=True)
        cols.append(idxv)
        prev = idxv
    cols.append(jnp.zeros((TILE, NBR_PAD - K), jnp.int32))
    idx_ref[...] = jnp.concatenate(cols, axis=1)


def _knn(xr, xc):
    return pl.pallas_call(
        _knn_body,
        grid=(N // TILE,),
        in_specs=[
            pl.BlockSpec((TILE, 3), lambda i: (i, 0)),
            pl.BlockSpec((3, N), lambda i: (0, 0)),
        ],
        out_specs=pl.BlockSpec((TILE, NBR_PAD), lambda i: (i, 0)),
        out_shape=jax.ShapeDtypeStruct((N, NBR_PAD), jnp.int32),
        scratch_shapes=[pltpu.VMEM((TILE, N), jnp.float32)],
    )(xr, xc)


def _sc_gather(table, idx2d, nrows, dcols):
    nw, ch = 32, 128
    per_w = nrows // nw
    n_ch = per_w // ch
    mesh = plsc.VectorSubcoreMesh(core_axis_name="c", subcore_axis_name="s")

    @functools.partial(
        pl.kernel, mesh=mesh,
        out_type=jax.ShapeDtypeStruct((nrows, dcols), jnp.float32),
        scratch_types=[
            pltpu.VMEM((n_ch, ch), jnp.int32),
            pltpu.VMEM((ch, dcols), jnp.float32),
            pltpu.SemaphoreType.DMA,
        ],
    )
    def k(table_hbm, idx_hbm, out_hbm, idx_v, rows_v, sem):
        wid = lax.axis_index("s") * 2 + lax.axis_index("c")
        pltpu.sync_copy(idx_hbm.at[pl.ds(wid * n_ch, n_ch)], idx_v)

        def body(i, carry):
            pltpu.async_copy(table_hbm.at[idx_v.at[i]], rows_v, sem).wait()
            pltpu.sync_copy(rows_v,
                            out_hbm.at[pl.ds(wid * per_w + i * ch, ch)])
            return carry
        jax.lax.fori_loop(0, n_ch, body, 0)

    return k(table, idx2d)


def _pconv_body(idx_ref, xyzpad_ref, table_ref, w0_ref, b0_ref, w1_ref,
                b1_ref, w2_ref, b2_ref, lin3_ref, linb_ref,
                pre_ref, stats_ref, g_ref, *, dpad, cout):
    def loadrow(n, carry):
        for k in range(K):
            iv = idx_ref[n, k]
            g_ref[pl.ds(k * TILE + n, 1), :] = table_ref[pl.ds(iv, 1), :]
        return carry
    jax.lax.fori_loop(0, TILE, loadrow, 0)

    xyz9 = jnp.concatenate([xyzpad_ref[...]] * K, axis=0)
    g_ref[...] = g_ref[...] - xyz9

    mask8 = (jax.lax.broadcasted_iota(jnp.int32, (1, 8), 1) < 3)
    g8 = jnp.where(mask8, g_ref[:, 0:8], 0.0)
    h = jnp.maximum(
        jnp.dot(g8, w0_ref[...], preferred_element_type=jnp.float32)
        + b0_ref[...], 0.0)
    h = jnp.maximum(
        jnp.dot(h, w1_ref[...], preferred_element_type=jnp.float32)
        + b1_ref[...], 0.0)
    w = jnp.maximum(
        jnp.dot(h, w2_ref[...], preferred_element_type=jnp.float32)
        + b2_ref[...], 0.0)

    pre = linb_ref[...]
    for wch in range(16):
        accw = jnp.zeros((TILE, dpad), jnp.float32)
        for k in range(K):
            accw = accw + (w[k * TILE:(k + 1) * TILE, wch:wch + 1]
                           * g_ref[k * TILE:(k + 1) * TILE, :])
        pre = pre + jnp.dot(accw, lin3_ref[wch],
                            preferred_element_type=jnp.float32)
    pre_ref[...] = pre

    @pl.when(pl.program_id(0) == 0)
    def _():
        stats_ref[...] = jnp.zeros_like(stats_ref)
    stats_ref[0:1, :] = stats_ref[0:1, :] + jnp.sum(pre, axis=0, keepdims=True)
    stats_ref[1:2, :] = stats_ref[1:2, :] + jnp.sum(pre * pre, axis=0,
                                                    keepdims=True)


def _pconv(idx, xyzpad, table, w0, b0, w1, b1, w2, b2, lin3, linb, dpad, cout):
    return pl.pallas_call(
        functools.partial(_pconv_body, dpad=dpad, cout=cout),
        grid=(N // TILE,),
        in_specs=[
            pl.BlockSpec((TILE, NBR_PAD), lambda i: (i, 0),
                         memory_space=pltpu.SMEM),
            pl.BlockSpec((TILE, dpad), lambda i: (i, 0)),
            pl.BlockSpec((N, dpad), lambda i: (0, 0)),
            pl.BlockSpec((8, 8), lambda i: (0, 0)),
            pl.BlockSpec((1, 8), lambda i: (0, 0)),
            pl.BlockSpec((8, 8), lambda i: (0, 0)),
            pl.BlockSpec((1, 8), lambda i: (0, 0)),
            pl.BlockSpec((8, 16), lambda i: (0, 0)),
            pl.BlockSpec((1, 16), lambda i: (0, 0)),
            pl.BlockSpec((16, dpad, cout), lambda i: (0, 0, 0)),
            pl.BlockSpec((1, cout), lambda i: (0, 0)),
        ],
        out_specs=[
            pl.BlockSpec((TILE, cout), lambda i: (i, 0)),
            pl.BlockSpec((8, cout), lambda i: (0, 0)),
        ],
        out_shape=[
            jax.ShapeDtypeStruct((N, cout), jnp.float32),
            jax.ShapeDtypeStruct((8, cout), jnp.float32),
        ],
        scratch_shapes=[pltpu.VMEM((TILE * K, dpad), jnp.float32)],
    )(idx, xyzpad, table, w0, b0, w1, b1, w2, b2, lin3, linb)


def _bn_body(pre_ref, stats_ref, g_ref, b_ref, out_ref):
    m = stats_ref[0:1, :] / N
    v = stats_ref[1:2, :] / N - m * m
    scale = g_ref[...] * jax.lax.rsqrt(v + EPS)
    out_ref[...] = _lrelu((pre_ref[...] - m) * scale + b_ref[...])


def _bn(pre, stats, gamma, beta, cout):
    return pl.pallas_call(
        _bn_body,
        grid=(N // T2,),
        in_specs=[
            pl.BlockSpec((T2, cout), lambda i: (i, 0)),
            pl.BlockSpec((8, cout), lambda i: (0, 0)),
            pl.BlockSpec((1, cout), lambda i: (0, 0)),
            pl.BlockSpec((1, cout), lambda i: (0, 0)),
        ],
        out_specs=pl.BlockSpec((T2, cout), lambda i: (i, 0)),
        out_shape=jax.ShapeDtypeStruct((N, cout), jnp.float32),
    )(pre, stats, gamma, beta)


def _tail_body(pre_ref, stats_ref, g_ref, b_ref, flow_ref, m0_ref, m0b_ref,
               m1_ref, m1b_ref, fc_ref, fcb_ref, np_ref, fl_ref):
    m = stats_ref[0:1, :] / N
    v = stats_ref[1:2, :] / N - m * m
    scale = g_ref[...] * jax.lax.rsqrt(v + EPS)
    x = _lrelu((pre_ref[...] - m) * scale + b_ref[...])
    h = _lrelu(jnp.dot(x, m0_ref[...], preferred_element_type=jnp.float32)
               + m0b_ref[...])
    h2 = _lrelu(jnp.dot(h, m1_ref[...], preferred_element_type=jnp.float32)
                + m1b_ref[...])
    fl = jnp.dot(h2, fc_ref[...], preferred_element_type=jnp.float32) \
        + fcb_ref[...]
    fl = jnp.clip(fl, -200.0, 200.0) + flow_ref[...]
    np_ref[...] = h2
    fl_ref[...] = fl


def _tail(pre, stats, gamma, beta, flowp, m0, m0b, m1, m1b, fc, fcb):
    return pl.pallas_call(
        _tail_body,
        grid=(N // T2,),
        in_specs=[
            pl.BlockSpec((T2, 128), lambda i: (i, 0)),
            pl.BlockSpec((8, 128), lambda i: (0, 0)),
            pl.BlockSpec((1, 128), lambda i: (0, 0)),
            pl.BlockSpec((1, 128), lambda i: (0, 0)),
            pl.BlockSpec((T2, 128), lambda i: (i, 0)),
            pl.BlockSpec((128, 128), lambda i: (0, 0)),
            pl.BlockSpec((1, 128), lambda i: (0, 0)),
            pl.BlockSpec((128, 64), lambda i: (0, 0)),
            pl.BlockSpec((1, 64), lambda i: (0, 0)),
            pl.BlockSpec((64, 128), lambda i: (0, 0)),
            pl.BlockSpec((1, 128), lambda i: (0, 0)),
        ],
        out_specs=[
            pl.BlockSpec((T2, 64), lambda i: (i, 0)),
            pl.BlockSpec((T2, 128), lambda i: (i, 0)),
        ],
        out_shape=[
            jax.ShapeDtypeStruct((N, 64), jnp.float32),
            jax.ShapeDtypeStruct((N, 128), jnp.float32),
        ],
    )(pre, stats, gamma, beta, flowp, m0, m0b, m1, m1b, fc, fcb)


def _prep_lin3(lin_w, d_in, dpad):
    c = lin_w.shape[0]
    l3 = lin_w.reshape(c, 3 + d_in, 16).transpose(2, 1, 0)
    return jnp.pad(l3, ((0, 0), (0, dpad - (3 + d_in)), (0, 0)))


def kernel(xyz, feats, cost_volume, flow, pc0_wn_w0, pc0_wn_b0, pc0_wn_w1,
           pc0_wn_b1, pc0_wn_w2, pc0_wn_b2, pc0_lin_w, pc0_lin_b, pc0_bn_g,
           pc0_bn_b, pc1_wn_w0, pc1_wn_b0, pc1_wn_w1, pc1_wn_b1, pc1_wn_w2,
           pc1_wn_b2, pc1_lin_w, pc1_lin_b, pc1_bn_g, pc1_bn_b, mlp0_w,
           mlp0_b, mlp1_w, mlp1_b, fc_w, fc_b):
    xc = xyz[0]
    xr = xc.T
    idx = _knn(xr, xc)

    pts0 = jnp.concatenate([feats, cost_volume], axis=1)[0].T
    d0, dpad0 = 192, 208
    table0 = jnp.pad(jnp.concatenate([xr, pts0], axis=1),
                     ((0, 0), (0, dpad0 - 3 - d0)))
    xyzpad0 = jnp.pad(xr, ((0, 0), (0, dpad0 - 3)))

    def wn_prep(w0, b0, w1, b1, w2, b2):
        w0p = jnp.zeros((8, 8), jnp.float32).at[0:3, :].set(w0.T)
        return (w0p, b0.reshape(1, 8), w1.T, b1.reshape(1, 8), w2.T,
                b2.reshape(1, 16))

    wn0 = wn_prep(pc0_wn_w0, pc0_wn_b0, pc0_wn_w1, pc0_wn_b1, pc0_wn_w2,
                  pc0_wn_b2)
    lin3_0 = _prep_lin3(pc0_lin_w, d0, dpad0)
    pre0, stats0 = _pconv(idx, xyzpad0, table0, *wn0, lin3_0,
                          pc0_lin_b.reshape(1, -1), dpad0, 128)
    pts1 = _bn(pre0, stats0, pc0_bn_g.reshape(1, -1),
               pc0_bn_b.reshape(1, -1), 128)

    d1, dpad1 = 128, 144
    table1 = jnp.pad(jnp.concatenate([xr, pts1], axis=1),
                     ((0, 0), (0, dpad1 - 3 - d1)))
    xyzpad1 = jnp.pad(xr, ((0, 0), (0, dpad1 - 3)))
    wn1 = wn_prep(pc1_wn_w0, pc1_wn_b0, pc1_wn_w1, pc1_wn_b1, pc1_wn_w2,
                  pc1_wn_b2)
    lin3_1 = _prep_lin3(pc1_lin_w, d1, dpad1)
    pre1, stats1 = _pconv(idx, xyzpad1, table1, *wn1, lin3_1,
                          pc1_lin_b.reshape(1, -1), dpad1, 128)

    flowp = jnp.pad(flow[0].T, ((0, 0), (0, 125)))
    fcp = jnp.pad(fc_w.T, ((0, 0), (0, 125)))
    fcbp = jnp.pad(fc_b, (0, 125)).reshape(1, 128)
    newp, flp = _tail(pre1, stats1, pc1_bn_g.reshape(1, -1),
                      pc1_bn_b.reshape(1, -1), flowp, mlp0_w.T,
                      mlp0_b.reshape(1, -1), mlp1_w.T, mlp1_b.reshape(1, -1),
                      fcp, fcbp)
    new_points = newp.T[None, :, :]
    flow_out = flp[:, 0:3].T[None, :, :]
    return new_points, flow_out
